# baseline (speedup 1.0000x reference)
import functools

import jax
import jax.numpy as jnp
from jax import lax
from jax.experimental import pallas as pl
from jax.experimental.pallas import tpu as pltpu
from jax.experimental.pallas import tpu_sc as plsc

N = 10000
E = 160000
ETOT = E + N
NW = 32
G = 48
NG = 111
EPT = NG * G
EPAD = NW * EPT
BN = 400
F32 = jnp.float32
I32 = jnp.int32


def _sc_mesh():
    return plsc.VectorSubcoreMesh(core_axis_name="c", subcore_axis_name="s")


def _make_edge_w(ha):
    out_type = (
        jax.ShapeDtypeStruct((ha, NW, 1, EPT), F32),
        jax.ShapeDtypeStruct((4 * N,), F32),
        jax.ShapeDtypeStruct((4 * N,), F32),
    )
    scratch = [
        pltpu.VMEM((NG, G), I32),
        pltpu.VMEM((NG, G), I32),
        pltpu.VMEM((2, G, 8), F32),
        pltpu.VMEM((2, G, 8), F32),
        pltpu.VMEM((ha, 1, EPT), F32),
        pltpu.VMEM((2, 2, 96), F32),
        pltpu.VMEM((2, 2, 96), I32),
        pltpu.VMEM_SHARED((4 * N,), F32),
        pltpu.SemaphoreType.DMA((2,)),
        pltpu.SemaphoreType.DMA((2,)),
        pltpu.SemaphoreType.DMA((2,)),
    ]

    @functools.partial(
        pl.kernel, mesh=_sc_mesh(), out_type=out_type, scratch_types=scratch,
        compiler_params=pltpu.CompilerParams(
            needs_layout_passes=False, use_tc_tiling_on_sc=False))
    def k(s_hbm, d_hbm, asad_hbm, z4_hbm, w_hbm, den0_hbm, den1_hbm,
          s_v, d_v, srows, drows, wa, wst, ist, den_sp, sem0, sem1, ssem):
        cid = lax.axis_index("c")
        sid = lax.axis_index("s")
        wid = sid * 2 + cid
        pltpu.sync_copy(s_hbm.at[wid], s_v)
        pltpu.sync_copy(d_hbm.at[wid], d_v)

        @pl.when(sid == 0)
        def _():
            pltpu.sync_copy(z4_hbm, den_sp)

        plsc.subcore_barrier()
        iota = lax.iota(I32, 16)

        def issue(g, p):
            pltpu.async_copy(asad_hbm.at[s_v.at[g]], srows.at[p], sem0.at[p])
            pltpu.async_copy(asad_hbm.at[d_v.at[g]], drows.at[p], sem1.at[p])

        def wait_scatter(p):
            for r in range(2):
                pltpu.make_async_copy(
                    wst.at[p, r], den_sp.at[ist.at[p, r]], ssem.at[p]).wait()

        issue(0, 0)

        def body(g, carry):
            p = lax.rem(g, 2)

            @pl.when(g + 1 < NG)
            def _():
                @pl.when(g >= 1)
                def _():
                    wait_scatter(1 - p)
                issue(g + 1, 1 - p)

            pltpu.make_async_copy(
                asad_hbm.at[s_v.at[g]], srows.at[p], sem0.at[p]).wait()
            pltpu.make_async_copy(
                asad_hbm.at[d_v.at[g]], drows.at[p], sem1.at[p]).wait()
            for j in range(3):
                d16 = d_v[g, pl.ds(j * 16, 16)]
                e16 = j * 16 + iota
                valid = (wid * EPT + g * G + j * 16 + iota) < ETOT
                for h in range(4):
                    slot = (j * 4 + h) * 16
                    row, col = divmod(slot, 96)
                    if h < ha:
                        a_s = plsc.load_gather(
                            srows, [jnp.full((16,), p, I32), e16,
                                    jnp.full((16,), h, I32)])
                        a_d = plsc.load_gather(
                            drows, [jnp.full((16,), p, I32), e16,
                                    jnp.full((16,), 4 + h, I32)])
                        a = a_s + a_d
                        a = jnp.where(a >= 0, a, 0.2 * a)
                        w = jnp.where(valid, jnp.exp(a), 0.0)
                        wa[h, 0, pl.ds(g * G + j * 16, 16)] = w
                    else:
                        w = jnp.zeros((16,), F32)
                    wst[p, row, pl.ds(col, 16)] = w
                    ist[p, row, pl.ds(col, 16)] = d16 * 4 + h
            for r in range(2):
                pltpu.async_copy(wst.at[p, r], den_sp.at[ist.at[p, r]],
                                 ssem.at[p], add=True)
            return carry

        lax.fori_loop(0, NG, body, 0)
        wait_scatter(lax.rem(NG - 1, 2))
        wait_scatter(lax.rem(NG, 2))
        plsc.subcore_barrier()

        @pl.when((cid == 0) & (sid == 0))
        def _():
            pltpu.sync_copy(den_sp, den0_hbm)

        @pl.when((cid == 1) & (sid == 0))
        def _():
            pltpu.sync_copy(den_sp, den1_hbm)

        for h in range(ha):
            pltpu.sync_copy(wa.at[h], w_hbm.at[h, wid])

    return k


def _make_agg(nchunk):
    nhalf = nchunk // 2
    out_type = tuple(
        jax.ShapeDtypeStruct((N, 128), F32) for _ in range(nchunk))
    scratch = [
        pltpu.VMEM((NG, G), I32),
        pltpu.VMEM((NG, G), I32),
        pltpu.VMEM((1, EPT), F32),
        pltpu.VMEM((5, G, 128), F32),
        pltpu.VMEM_SHARED((N, 128), F32),
        pltpu.SemaphoreType.DMA((5,)),
        pltpu.SemaphoreType.DMA((5,)),
    ]

    @functools.partial(
        pl.kernel, mesh=_sc_mesh(), out_type=out_type, scratch_types=scratch,
        compiler_params=pltpu.CompilerParams(
            needs_layout_passes=False, use_tc_tiling_on_sc=False))
    def k(*refs):
        (s_hbm, d_hbm, w_hbm, z_hbm) = refs[:4]
        h_refs = refs[4:4 + nchunk]
        agg_refs = refs[4 + nchunk:4 + 2 * nchunk]
        (s_v, d_v, w_t, rows, acc_sp, gsem, ssem) = refs[4 + 2 * nchunk:]
        cid = lax.axis_index("c")
        sid = lax.axis_index("s")

        sw = (N // (16 * 8)) * 8
        tail = N - 16 * sw

        def striped(src, dst):
            pltpu.sync_copy(src.at[pl.ds(sid * sw, sw)],
                            dst.at[pl.ds(sid * sw, sw)])
            if tail:
                @pl.when(sid == 15)
                def _():
                    pltpu.sync_copy(src.at[pl.ds(16 * sw, tail)],
                                    dst.at[pl.ds(16 * sw, tail)])

        for c_local in range(nhalf):
            striped(z_hbm, acc_sp)
            plsc.subcore_barrier()

            def edge_loop(href):
                def issue(g, p):
                    pltpu.async_copy(href.at[s_v.at[g]], rows.at[p],
                                     gsem.at[p])

                def wait_scatter(p):
                    pltpu.make_async_copy(
                        rows.at[p], acc_sp.at[d_v.at[0]], ssem.at[p]).wait()

                def qbody(q, carry):
                    wq = sid * 2 + q
                    pltpu.sync_copy(s_hbm.at[wq], s_v)
                    pltpu.sync_copy(d_hbm.at[wq], d_v)
                    pltpu.sync_copy(w_hbm.at[c_local, wq], w_t)
                    issue(0, 0)
                    issue(1, 1)
                    issue(2, 2)

                    def body(g, carry2):
                        p = lax.rem(g, 5)
                        pnx = lax.rem(g + 3, 5)

                        @pl.when(g + 3 < NG)
                        def _():
                            @pl.when(g >= 2)
                            def _():
                                wait_scatter(pnx)
                            issue(g + 3, pnx)

                        pltpu.make_async_copy(
                            href.at[s_v.at[g]], rows.at[p], gsem.at[p]).wait()

                        def rbody(r, rc):
                            for u in range(4):
                                ri = r * 4 + u
                                a16 = plsc.load_gather(
                                    w_t, [jnp.full((16,), 0, I32),
                                          jnp.full((16,), g * G + ri, I32)])
                                for kk in range(8):
                                    sl = pl.ds(kk * 16, 16)
                                    rows[p, ri, sl] = a16 * rows[p, ri, sl]
                            return rc

                        lax.fori_loop(0, G // 4, rbody, 0)
                        pltpu.async_copy(rows.at[p], acc_sp.at[d_v.at[g]],
                                         ssem.at[p], add=True)
                        return carry2

                    lax.fori_loop(0, NG, body, 0)
                    for b in range(5):
                        wait_scatter(lax.rem(NG - 5 + b, 5))
                    return carry

                lax.fori_loop(0, 2, qbody, 0)

            @pl.when(cid == 0)
            def _():
                edge_loop(h_refs[2 * c_local])

            @pl.when(cid == 1)
            def _():
                edge_loop(h_refs[2 * c_local + 1])

            plsc.subcore_barrier()

            @pl.when(cid == 0)
            def _():
                striped(acc_sp, agg_refs[2 * c_local])

            @pl.when(cid == 1)
            def _():
                striped(acc_sp, agg_refs[2 * c_local + 1])

            plsc.subcore_barrier()

    return k


def _tc1_body(x_ref, w_ref, a_ref, *outs):
    j = pl.program_id(1)
    h = jnp.dot(x_ref[...], w_ref[...], preferred_element_type=F32)
    for c in range(8):
        @pl.when(j == c)
        def _():
            outs[c][...] = h
    part = jnp.dot(h, a_ref[...], preferred_element_type=F32)
    asad = outs[8]

    @pl.when(j == 0)
    def _():
        asad[...] = part

    @pl.when(j > 0)
    def _():
        asad[...] = asad[...] + part


def _tc2_body(*refs):
    p = refs[:8]
    b_ref, w2_ref, a2_ref, d0_ref, d1_ref = refs[8:13]
    h2 = refs[13:15]
    asad2 = refs[15]
    xs = refs[16]
    j2 = pl.program_id(1)
    kk = pl.program_id(2)
    for c in range(8):
        @pl.when(kk == c)
        def _():
            hd = c // 2
            r = 1.0 / (d0_ref[...][:, hd:hd + 1]
                       + d1_ref[...][:, hd:hd + 1] + 1e-16)
            xs[...] = p[c][...] * r
    e = xs[...] + b_ref[0]
    x2 = jnp.where(e > 0, e, jnp.exp(e) - 1.0)
    mm = jnp.dot(x2, w2_ref[...], preferred_element_type=F32)
    for c2 in range(2):
        @pl.when(j2 == c2)
        def _():
            @pl.when(kk == 0)
            def _():
                h2[c2][...] = mm

            @pl.when(kk > 0)
            def _():
                h2[c2][...] = h2[c2][...] + mm

            @pl.when(kk == 7)
            def _():
                part = jnp.dot(h2[c2][...], a2_ref[...],
                               preferred_element_type=F32)

                @pl.when(j2 == 0)
                def _():
                    asad2[...] = part

                @pl.when(j2 == 1)
                def _():
                    asad2[...] = asad2[...] + part


def _tc3_body(c0_ref, c1_ref, b_ref, d0_ref, d1_ref, o_ref):
    r = 1.0 / (d0_ref[...][:, 0:1] + d1_ref[...][:, 0:1] + 1e-16)
    o_ref[...] = (
        jnp.concatenate([c0_ref[...] * r, c1_ref[...] * r], axis=1)
        + b_ref[...])


def _tc1(x, w1, a1m):
    grid = (N // BN, 8)
    return pl.pallas_call(
        _tc1_body,
        grid=grid,
        in_specs=[
            pl.BlockSpec((BN, 256), lambda i, j: (i, 0)),
            pl.BlockSpec((256, 128), lambda i, j: (0, j)),
            pl.BlockSpec((128, 8), lambda i, j: (j, 0)),
        ],
        out_specs=[pl.BlockSpec((BN, 128), lambda i, j: (i, 0))
                   for _ in range(8)]
        + [pl.BlockSpec((BN, 8), lambda i, j: (i, 0))],
        out_shape=[jax.ShapeDtypeStruct((N, 128), F32) for _ in range(8)]
        + [jax.ShapeDtypeStruct((N, 8), F32)],
    )(x, w1, a1m)


def _tc2(aggs, b1r, w2, a2m, den0, den1):
    grid = (N // BN, 2, 8)
    return pl.pallas_call(
        _tc2_body,
        grid=grid,
        in_specs=[pl.BlockSpec((BN, 128), lambda i, j, k: (i, 0))
                  for _ in range(8)]
        + [
            pl.BlockSpec((1, 1, 128), lambda i, j, k: (k, 0, 0)),
            pl.BlockSpec((128, 128), lambda i, j, k: (k, j)),
            pl.BlockSpec((128, 8), lambda i, j, k: (j, 0)),
            pl.BlockSpec((BN, 4), lambda i, j, k: (i, 0)),
            pl.BlockSpec((BN, 4), lambda i, j, k: (i, 0)),
        ],
        out_specs=[pl.BlockSpec((BN, 128), lambda i, j, k: (i, 0))
                   for _ in range(2)]
        + [pl.BlockSpec((BN, 8), lambda i, j, k: (i, 0))],
        out_shape=[jax.ShapeDtypeStruct((N, 128), F32) for _ in range(2)]
        + [jax.ShapeDtypeStruct((N, 8), F32)],
        scratch_shapes=[pltpu.VMEM((BN, 128), F32)],
    )(*aggs, b1r, w2, a2m, den0, den1)


def _tc3(c0, c1, b2r, den0, den1):
    grid = (N // BN,)
    return pl.pallas_call(
        _tc3_body,
        grid=grid,
        in_specs=[
            pl.BlockSpec((BN, 128), lambda i: (i, 0)),
            pl.BlockSpec((BN, 128), lambda i: (i, 0)),
            pl.BlockSpec((1, 256), lambda i: (0, 0)),
            pl.BlockSpec((BN, 4), lambda i: (i, 0)),
            pl.BlockSpec((BN, 4), lambda i: (i, 0)),
        ],
        out_specs=pl.BlockSpec((BN, 256), lambda i: (i, 0)),
        out_shape=jax.ShapeDtypeStruct((N, 256), F32),
    )(c0, c1, b2r, den0, den1)


def _block_diag_att(att_s, att_d, heads):
    eye = jnp.eye(heads, 4, dtype=F32)
    a_s = (att_s[:, :, None] * eye[:, None, :]).reshape(-1, 4)
    a_d = (att_d[:, :, None] * eye[:, None, :]).reshape(-1, 4)
    return jnp.concatenate([a_s, a_d], axis=1)


def kernel(x, edge_index, W1, att_src1, att_dst1, bias1,
           W2, att_src2, att_dst2, bias2):
    src = edge_index[0].astype(I32)
    dst = edge_index[1].astype(I32)
    loop = jnp.arange(N, dtype=I32)
    pad = jnp.zeros((EPAD - ETOT,), I32)
    s_all = jnp.concatenate([src, loop, pad]).reshape(NW, NG, G)
    d_all = jnp.concatenate([dst, loop, pad]).reshape(NW, NG, G)
    z4 = jnp.zeros((4 * N,), F32)
    z128 = jnp.zeros((N, 128), F32)
    a1m = _block_diag_att(att_src1[0], att_dst1[0], 4)
    a2m = _block_diag_att(att_src2[0], att_dst2[0], 1)

    *h1c, asad1 = _tc1(x, W1, a1m)
    w1, den0, den1 = _make_edge_w(4)(s_all, d_all, asad1, z4)
    agg1 = _make_agg(8)(s_all, d_all, w1, z128, *h1c)
    h2c0, h2c1, asad2 = _tc2(agg1, bias1.reshape(8, 1, 128), W2, a2m,
                             den0.reshape(N, 4), den1.reshape(N, 4))
    w2, e0, e1 = _make_edge_w(1)(s_all, d_all, asad2, z4)
    g2 = _make_agg(2)(s_all, d_all, w2, z128, h2c0, h2c1)
    return _tc3(g2[0], g2[1], bias2.reshape(1, 256),
                e0.reshape(N, 4), e1.reshape(N, 4))


# speedup vs baseline: 2.0284x; 2.0284x over previous
import functools

import jax
import jax.numpy as jnp
from jax import lax
from jax.experimental import pallas as pl
from jax.experimental.pallas import tpu as pltpu
from jax.experimental.pallas import tpu_sc as plsc

N = 10000
E = 160000
ETOT = E + N
NW = 32
G = 48
NG = 111
EPT = NG * G
EPAD = NW * EPT
BN = 400
F32 = jnp.float32
I32 = jnp.int32


def _sc_mesh():
    return plsc.VectorSubcoreMesh(core_axis_name="c", subcore_axis_name="s")


def _make_edge_w(ha):
    out_type = (
        jax.ShapeDtypeStruct((ha, NW, 1, EPT), F32),
        jax.ShapeDtypeStruct((4 * N,), F32),
        jax.ShapeDtypeStruct((4 * N,), F32),
    )
    scratch = [
        pltpu.VMEM((NG, G), I32),
        pltpu.VMEM((NG, G), I32),
        pltpu.VMEM((2, G, 8), F32),
        pltpu.VMEM((2, G, 8), F32),
        pltpu.VMEM((ha, 1, EPT), F32),
        pltpu.VMEM((2, 2, 96), F32),
        pltpu.VMEM((2, 2, 96), I32),
        pltpu.VMEM_SHARED((4 * N,), F32),
        pltpu.SemaphoreType.DMA((2,)),
        pltpu.SemaphoreType.DMA((2,)),
        pltpu.SemaphoreType.DMA((2,)),
    ]

    @functools.partial(
        pl.kernel, mesh=_sc_mesh(), out_type=out_type, scratch_types=scratch,
        compiler_params=pltpu.CompilerParams(
            needs_layout_passes=False, use_tc_tiling_on_sc=False))
    def k(s_hbm, d_hbm, asad_hbm, z4_hbm, w_hbm, den0_hbm, den1_hbm,
          s_v, d_v, srows, drows, wa, wst, ist, den_sp, sem0, sem1, ssem):
        cid = lax.axis_index("c")
        sid = lax.axis_index("s")
        wid = sid * 2 + cid
        pltpu.sync_copy(s_hbm.at[wid], s_v)
        pltpu.sync_copy(d_hbm.at[wid], d_v)

        @pl.when(sid == 0)
        def _():
            pltpu.sync_copy(z4_hbm, den_sp)

        plsc.subcore_barrier()
        iota = lax.iota(I32, 16)

        def issue(g, p):
            pltpu.async_copy(asad_hbm.at[s_v.at[g]], srows.at[p], sem0.at[p])
            pltpu.async_copy(asad_hbm.at[d_v.at[g]], drows.at[p], sem1.at[p])

        def wait_scatter(p):
            for r in range(2):
                pltpu.make_async_copy(
                    wst.at[p, r], den_sp.at[ist.at[p, r]], ssem.at[p]).wait()

        issue(0, 0)

        def body(g, carry):
            p = lax.rem(g, 2)

            @pl.when(g + 1 < NG)
            def _():
                @pl.when(g >= 1)
                def _():
                    wait_scatter(1 - p)
                issue(g + 1, 1 - p)

            pltpu.make_async_copy(
                asad_hbm.at[s_v.at[g]], srows.at[p], sem0.at[p]).wait()
            pltpu.make_async_copy(
                asad_hbm.at[d_v.at[g]], drows.at[p], sem1.at[p]).wait()
            for j in range(3):
                d16 = d_v[g, pl.ds(j * 16, 16)]
                e16 = j * 16 + iota
                valid = (wid * EPT + g * G + j * 16 + iota) < ETOT
                for h in range(4):
                    slot = (j * 4 + h) * 16
                    row, col = divmod(slot, 96)
                    if h < ha:
                        a_s = plsc.load_gather(
                            srows, [jnp.full((16,), p, I32), e16,
                                    jnp.full((16,), h, I32)])
                        a_d = plsc.load_gather(
                            drows, [jnp.full((16,), p, I32), e16,
                                    jnp.full((16,), 4 + h, I32)])
                        a = a_s + a_d
                        a = jnp.where(a >= 0, a, 0.2 * a)
                        w = jnp.where(valid, jnp.exp(a), 0.0)
                        wa[h, 0, pl.ds(g * G + j * 16, 16)] = w
                    else:
                        w = jnp.zeros((16,), F32)
                    wst[p, row, pl.ds(col, 16)] = w
                    ist[p, row, pl.ds(col, 16)] = d16 * 4 + h
            for r in range(2):
                pltpu.async_copy(wst.at[p, r], den_sp.at[ist.at[p, r]],
                                 ssem.at[p], add=True)
            return carry

        lax.fori_loop(0, NG, body, 0)
        wait_scatter(lax.rem(NG - 1, 2))
        wait_scatter(lax.rem(NG, 2))
        plsc.subcore_barrier()

        @pl.when((cid == 0) & (sid == 0))
        def _():
            pltpu.sync_copy(den_sp, den0_hbm)

        @pl.when((cid == 1) & (sid == 0))
        def _():
            pltpu.sync_copy(den_sp, den1_hbm)

        for h in range(ha):
            pltpu.sync_copy(wa.at[h], w_hbm.at[h, wid])

    return k


def _make_agg(nchunk):
    nhalf = nchunk // 2
    out_type = tuple(
        jax.ShapeDtypeStruct((N, 128), F32) for _ in range(nchunk))
    scratch = [
        pltpu.VMEM((NG, G), I32),
        pltpu.VMEM((NG, G), I32),
        pltpu.VMEM((1, EPT), F32),
        pltpu.VMEM((4, G, 128), F32),
        pltpu.VMEM_SHARED((N, 128), F32),
        pltpu.SemaphoreType.DMA((4,)),
        pltpu.SemaphoreType.DMA((4,)),
    ]

    @functools.partial(
        pl.kernel, mesh=_sc_mesh(), out_type=out_type, scratch_types=scratch,
        compiler_params=pltpu.CompilerParams(
            needs_layout_passes=False, use_tc_tiling_on_sc=False))
    def k(*refs):
        (s_hbm, d_hbm, w_hbm, z_hbm) = refs[:4]
        h_refs = refs[4:4 + nchunk]
        agg_refs = refs[4 + nchunk:4 + 2 * nchunk]
        (s_v, d_v, w_t, rows, acc_sp, gsem, ssem) = refs[4 + 2 * nchunk:]
        cid = lax.axis_index("c")
        sid = lax.axis_index("s")

        sw = (N // (16 * 8)) * 8
        tail = N - 16 * sw

        def striped(src, dst):
            pltpu.sync_copy(src.at[pl.ds(sid * sw, sw)],
                            dst.at[pl.ds(sid * sw, sw)])
            if tail:
                @pl.when(sid == 15)
                def _():
                    pltpu.sync_copy(src.at[pl.ds(16 * sw, tail)],
                                    dst.at[pl.ds(16 * sw, tail)])

        for c_local in range(nhalf):
            striped(z_hbm, acc_sp)
            plsc.subcore_barrier()

            def edge_loop(href):
                def issue(g, p):
                    pltpu.async_copy(href.at[s_v.at[g]], rows.at[p],
                                     gsem.at[p])

                def wait_scatter(p):
                    pltpu.make_async_copy(
                        rows.at[p], acc_sp.at[d_v.at[0]], ssem.at[p]).wait()

                def qbody(q, carry):
                    wq = sid * 2 + q
                    pltpu.sync_copy(s_hbm.at[wq], s_v)
                    pltpu.sync_copy(d_hbm.at[wq], d_v)
                    pltpu.sync_copy(w_hbm.at[c_local, wq], w_t)
                    issue(0, 0)
                    issue(1, 1)

                    def body(g, carry2):
                        p = lax.rem(g, 4)
                        pnx = lax.rem(g + 2, 4)

                        @pl.when(g + 2 < NG)
                        def _():
                            @pl.when(g >= 2)
                            def _():
                                wait_scatter(pnx)
                            issue(g + 2, pnx)

                        pltpu.make_async_copy(
                            href.at[s_v.at[g]], rows.at[p], gsem.at[p]).wait()

                        def rbody(r, rc):
                            for u in range(4):
                                ri = r * 4 + u
                                a16 = plsc.load_gather(
                                    w_t, [jnp.full((16,), 0, I32),
                                          jnp.full((16,), g * G + ri, I32)])
                                for kk in range(8):
                                    sl = pl.ds(kk * 16, 16)
                                    rows[p, ri, sl] = a16 * rows[p, ri, sl]
                            return rc

                        lax.fori_loop(0, G // 4, rbody, 0)
                        pltpu.async_copy(rows.at[p], acc_sp.at[d_v.at[g]],
                                         ssem.at[p], add=True)
                        return carry2

                    lax.fori_loop(0, NG, body, 0)
                    for b in range(4):
                        wait_scatter(lax.rem(NG - 4 + b, 4))
                    return carry

                lax.fori_loop(0, 2, qbody, 0)

            @pl.when(cid == 0)
            def _():
                edge_loop(h_refs[2 * c_local])

            @pl.when(cid == 1)
            def _():
                edge_loop(h_refs[2 * c_local + 1])

            plsc.subcore_barrier()

            @pl.when(cid == 0)
            def _():
                striped(acc_sp, agg_refs[2 * c_local])

            @pl.when(cid == 1)
            def _():
                striped(acc_sp, agg_refs[2 * c_local + 1])

            plsc.subcore_barrier()

    return k


def _tc1_body(x_ref, w_ref, a_ref, *outs):
    j = pl.program_id(1)
    h = jnp.dot(x_ref[...], w_ref[...], preferred_element_type=F32)
    for c in range(8):
        @pl.when(j == c)
        def _():
            outs[c][...] = h
    part = jnp.dot(h, a_ref[...], preferred_element_type=F32)
    asad = outs[8]

    @pl.when(j == 0)
    def _():
        asad[...] = part

    @pl.when(j > 0)
    def _():
        asad[...] = asad[...] + part


def _tc2_body(*refs):
    p = refs[:8]
    b_ref, w2_ref, a2_ref, d0_ref, d1_ref = refs[8:13]
    h2 = refs[13:15]
    asad2 = refs[15]
    xs = refs[16]
    j2 = pl.program_id(1)
    kk = pl.program_id(2)
    for c in range(8):
        @pl.when(kk == c)
        def _():
            hd = c // 2
            r = 1.0 / (d0_ref[...][:, hd:hd + 1]
                       + d1_ref[...][:, hd:hd + 1] + 1e-16)
            xs[...] = p[c][...] * r
    e = xs[...] + b_ref[0]
    x2 = jnp.where(e > 0, e, jnp.exp(e) - 1.0)
    mm = jnp.dot(x2, w2_ref[...], preferred_element_type=F32)
    for c2 in range(2):
        @pl.when(j2 == c2)
        def _():
            @pl.when(kk == 0)
            def _():
                h2[c2][...] = mm

            @pl.when(kk > 0)
            def _():
                h2[c2][...] = h2[c2][...] + mm

            @pl.when(kk == 7)
            def _():
                part = jnp.dot(h2[c2][...], a2_ref[...],
                               preferred_element_type=F32)

                @pl.when(j2 == 0)
                def _():
                    asad2[...] = part

                @pl.when(j2 == 1)
                def _():
                    asad2[...] = asad2[...] + part


def _tc3_body(c0_ref, c1_ref, b_ref, d0_ref, d1_ref, o_ref):
    r = 1.0 / (d0_ref[...][:, 0:1] + d1_ref[...][:, 0:1] + 1e-16)
    o_ref[...] = (
        jnp.concatenate([c0_ref[...] * r, c1_ref[...] * r], axis=1)
        + b_ref[...])


def _tc1(x, w1, a1m):
    grid = (N // BN, 8)
    return pl.pallas_call(
        _tc1_body,
        grid=grid,
        in_specs=[
            pl.BlockSpec((BN, 256), lambda i, j: (i, 0)),
            pl.BlockSpec((256, 128), lambda i, j: (0, j)),
            pl.BlockSpec((128, 8), lambda i, j: (j, 0)),
        ],
        out_specs=[pl.BlockSpec((BN, 128), lambda i, j: (i, 0))
                   for _ in range(8)]
        + [pl.BlockSpec((BN, 8), lambda i, j: (i, 0))],
        out_shape=[jax.ShapeDtypeStruct((N, 128), F32) for _ in range(8)]
        + [jax.ShapeDtypeStruct((N, 8), F32)],
    )(x, w1, a1m)


def _tc2(aggs, b1r, w2, a2m, den0, den1):
    grid = (N // BN, 2, 8)
    return pl.pallas_call(
        _tc2_body,
        grid=grid,
        in_specs=[pl.BlockSpec((BN, 128), lambda i, j, k: (i, 0))
                  for _ in range(8)]
        + [
            pl.BlockSpec((1, 1, 128), lambda i, j, k: (k, 0, 0)),
            pl.BlockSpec((128, 128), lambda i, j, k: (k, j)),
            pl.BlockSpec((128, 8), lambda i, j, k: (j, 0)),
            pl.BlockSpec((BN, 4), lambda i, j, k: (i, 0)),
            pl.BlockSpec((BN, 4), lambda i, j, k: (i, 0)),
        ],
        out_specs=[pl.BlockSpec((BN, 128), lambda i, j, k: (i, 0))
                   for _ in range(2)]
        + [pl.BlockSpec((BN, 8), lambda i, j, k: (i, 0))],
        out_shape=[jax.ShapeDtypeStruct((N, 128), F32) for _ in range(2)]
        + [jax.ShapeDtypeStruct((N, 8), F32)],
        scratch_shapes=[pltpu.VMEM((BN, 128), F32)],
    )(*aggs, b1r, w2, a2m, den0, den1)


def _tc3(c0, c1, b2r, den0, den1):
    grid = (N // BN,)
    return pl.pallas_call(
        _tc3_body,
        grid=grid,
        in_specs=[
            pl.BlockSpec((BN, 128), lambda i: (i, 0)),
            pl.BlockSpec((BN, 128), lambda i: (i, 0)),
            pl.BlockSpec((1, 256), lambda i: (0, 0)),
            pl.BlockSpec((BN, 4), lambda i: (i, 0)),
            pl.BlockSpec((BN, 4), lambda i: (i, 0)),
        ],
        out_specs=pl.BlockSpec((BN, 256), lambda i: (i, 0)),
        out_shape=jax.ShapeDtypeStruct((N, 256), F32),
    )(c0, c1, b2r, den0, den1)


def _block_diag_att(att_s, att_d, heads):
    eye = jnp.eye(heads, 4, dtype=F32)
    a_s = (att_s[:, :, None] * eye[:, None, :]).reshape(-1, 4)
    a_d = (att_d[:, :, None] * eye[:, None, :]).reshape(-1, 4)
    return jnp.concatenate([a_s, a_d], axis=1)


def kernel(x, edge_index, W1, att_src1, att_dst1, bias1,
           W2, att_src2, att_dst2, bias2):
    src = edge_index[0].astype(I32)
    dst = edge_index[1].astype(I32)
    loop = jnp.arange(N, dtype=I32)
    pad = jnp.zeros((EPAD - ETOT,), I32)
    s_all = jnp.concatenate([src, loop, pad]).reshape(NW, NG, G)
    d_all = jnp.concatenate([dst, loop, pad]).reshape(NW, NG, G)
    z4 = jnp.zeros((4 * N,), F32)
    z128 = jnp.zeros((N, 128), F32)
    a1m = _block_diag_att(att_src1[0], att_dst1[0], 4)
    a2m = _block_diag_att(att_src2[0], att_dst2[0], 1)

    *h1c, asad1 = _tc1(x, W1, a1m)
    w1, den0, den1 = _make_edge_w(4)(s_all, d_all, asad1, z4)
    agg1 = _make_agg(8)(s_all, d_all, w1, z128, *h1c)
    h2c0, h2c1, asad2 = _tc2(agg1, bias1.reshape(8, 1, 128), W2, a2m,
                             den0.reshape(N, 4), den1.reshape(N, 4))
    w2, e0, e1 = _make_edge_w(1)(s_all, d_all, asad2, z4)
    g2 = _make_agg(2)(s_all, d_all, w2, z128, h2c0, h2c1)
    return _tc3(g2[0], g2[1], bias2.reshape(1, 256),
                e0.reshape(N, 4), e1.reshape(N, 4))
